# baseline (speedup 1.0000x reference)
import jax
import jax.numpy as jnp
from jax.experimental import pallas as pl
from jax.experimental.pallas import tpu as pltpu

Y = 64
SUB, LANE = 256, 128
NSLOT = 9


def _req_map(b, nread, ntgt, rows):
    return (b, 0, 0)


def _sig_map(b, nread, ntgt, rows):
    return (b, 0, 0)


def _out_map(b, nread, ntgt, rows):
    return (b, 0, 0)


def _hist_kernel(
    nread_ref, ntgt_ref, rows_ref, tab_hbm, req_ref, sig_ref, out_ref, buf, sems
):
    b = pl.program_id(0)
    nb = pl.num_programs(0)

    def issue(bt, slot):
        for j in range(NSLOT):

            @pl.when(j < nread_ref[bt])
            def _():
                pltpu.make_async_copy(
                    tab_hbm.at[bt, rows_ref[bt, j]],
                    buf.at[slot, j],
                    sems.at[slot, j],
                ).start()

    @pl.when(b == 0)
    def _():
        buf[...] = jnp.zeros(buf.shape, buf.dtype)
        issue(0, 0)

    @pl.when(b + 1 < nb)
    def _():
        issue(b + 1, (b + 1) % 2)

    for j in range(NSLOT):

        @pl.when(j < nread_ref[b])
        def _():
            pltpu.make_async_copy(
                tab_hbm.at[b, rows_ref[b, j]],
                buf.at[b % 2, j],
                sems.at[b % 2, j],
            ).wait()

    tab = buf[b % 2]
    req = req_ref[0]
    eq = (tab == req).astype(jnp.float32)
    cnt = jax.lax.dot_general(
        jnp.ones((1, tab.shape[0]), jnp.float32),
        eq,
        (((1,), (0,)), ((), ())),
        preferred_element_type=jnp.float32,
    )
    key = (
        cnt.reshape(SUB, LANE) * Y
        + sig_ref[0].astype(jnp.float32)
        - (ntgt_ref[b] * Y).astype(jnp.float32)
    )
    hist = jnp.stack(
        [jnp.sum((key == c).astype(jnp.float32)) for c in range(Y)]
    ).reshape(1, Y)
    z = jnp.maximum(jnp.sum(hist), 1.0)
    out_ref[0] = jnp.log(jnp.maximum(hist / z, 1e-9))


def kernel(tables, sigma, base_obs, actions, responses, t):
    B, V, N = tables.shape
    T = actions.shape[1]
    assert N == SUB * LANE

    actions = actions.astype(jnp.int32)
    responses = responses.astype(jnp.int32)
    base_obs = base_obs.astype(jnp.int32)

    active = jnp.arange(T, dtype=jnp.int32)[None, :] < t
    use_real = active & (actions != V)
    a_c = jnp.clip(actions, 0, V - 1)
    rows = jnp.concatenate(
        [jnp.zeros((B, 1), jnp.int32), jnp.where(use_real, a_c, 0)], axis=1
    )
    vals = jnp.concatenate(
        [base_obs[:, None], jnp.where(use_real, responses, base_obs[:, None])],
        axis=1,
    )

    BIG = jnp.int32(1 << 20)
    v_iota = jnp.arange(V, dtype=jnp.int32)
    hit = rows[:, None, :] == v_iota[None, :, None]
    vmin = jnp.min(jnp.where(hit, vals[:, None, :], BIG), axis=2)
    vmax = jnp.max(jnp.where(hit, vals[:, None, :], -BIG), axis=2)
    con = jnp.any(hit, axis=2)
    feasible = jnp.all(~con | (vmin == vmax), axis=1)
    ncon = jnp.sum(con.astype(jnp.int32), axis=1)

    sort_key = jnp.where(con, v_iota[None, :], jnp.int32(2 * V))
    slot_rows_full = jnp.sort(sort_key, axis=1)
    slot_valid = jnp.arange(V, dtype=jnp.int32)[None, :] < ncon[:, None]
    slot_rows = jnp.where(slot_valid, slot_rows_full, 0)
    req_full = jnp.where(con, vmin, -1).astype(jnp.int32)
    slot_vals = jnp.where(
        slot_valid, jnp.take_along_axis(req_full, slot_rows, axis=1), -1
    ).astype(jnp.int32)

    nread = jnp.where(feasible, ncon, 0).astype(jnp.int32)
    ntgt = jnp.where(feasible, ncon, V + 1).astype(jnp.int32)
    slot_rows9 = slot_rows[:, :NSLOT]

    s3 = sigma.reshape(B, SUB, LANE)
    req3 = slot_vals[:, :, None]

    grid_spec = pltpu.PrefetchScalarGridSpec(
        num_scalar_prefetch=3,
        grid=(B,),
        in_specs=(
            pl.BlockSpec(memory_space=pl.ANY),
            pl.BlockSpec((1, V, 1), _req_map),
            pl.BlockSpec((1, SUB, LANE), _sig_map),
        ),
        out_specs=pl.BlockSpec((1, 1, Y), _out_map),
        scratch_shapes=[
            pltpu.VMEM((2, V, N), jnp.int32),
            pltpu.SemaphoreType.DMA((2, NSLOT)),
        ],
    )
    out = pl.pallas_call(
        _hist_kernel,
        grid_spec=grid_spec,
        out_shape=jax.ShapeDtypeStruct((B, 1, Y), jnp.float32),
        compiler_params=pltpu.CompilerParams(
            dimension_semantics=("arbitrary",)
        ),
    )(nread, ntgt, slot_rows9, tables, req3, s3)
    return out.reshape(B, Y)


# speedup vs baseline: 1.3157x; 1.3157x over previous
import jax
import jax.numpy as jnp
from jax.experimental import pallas as pl
from jax.experimental.pallas import tpu as pltpu

Y = 64
SUB, LANE = 256, 128
BB = 2


def kernel(tables, sigma, base_obs, actions, responses, t):
    B, V, N = tables.shape
    T = actions.shape[1]
    assert N == SUB * LANE and B % BB == 0

    actions = actions.astype(jnp.int32)
    responses = responses.astype(jnp.int32)
    base_obs = base_obs.astype(jnp.int32)

    active = jnp.arange(T, dtype=jnp.int32)[None, :] < t
    use_real = active & (actions != V)
    a_c = jnp.clip(actions, 0, V - 1)
    rows = jnp.concatenate(
        [jnp.zeros((B, 1), jnp.int32), jnp.where(use_real, a_c, 0)], axis=1
    )
    vals = jnp.concatenate(
        [base_obs[:, None], jnp.where(use_real, responses, base_obs[:, None])],
        axis=1,
    )

    BIG = jnp.int32(1 << 20)
    hit = rows[:, None, :] == jnp.arange(V, dtype=jnp.int32)[None, :, None]
    vmin = jnp.min(jnp.where(hit, vals[:, None, :], BIG), axis=2)
    vmax = jnp.max(jnp.where(hit, vals[:, None, :], -BIG), axis=2)
    con = jnp.any(hit, axis=2)
    req = jnp.where(con, vmin, -1).astype(jnp.int32)
    feasible = jnp.all(~con | (vmin == vmax), axis=1)
    ncon = jnp.where(
        feasible, jnp.sum(con.astype(jnp.int32), axis=1), V + 1
    ).astype(jnp.int32)

    s3 = sigma.reshape(B // BB, BB, SUB, LANE)
    req3 = req.reshape(B // BB, BB, V)[:, :, :, None]
    t4 = tables.reshape(B // BB, BB, V, N)

    grid_spec = pltpu.PrefetchScalarGridSpec(
        num_scalar_prefetch=1,
        grid=(B // BB,),
        in_specs=(
            pl.BlockSpec((1, BB, V, N), lambda g, s: (g, 0, 0, 0)),
            pl.BlockSpec((1, BB, V, 1), lambda g, s: (g, 0, 0, 0)),
            pl.BlockSpec((1, BB, SUB, LANE), lambda g, s: (g, 0, 0, 0)),
        ),
        out_specs=pl.BlockSpec((1, BB, 1, Y), lambda g, s: (g, 0, 0, 0)),
    )

    def body(ncon_ref, tab_ref, req_ref, sig_ref, out_ref):
        g = pl.program_id(0)
        for i in range(BB):
            tab = tab_ref[0, i]
            req = req_ref[0, i]
            eq = (tab == req).astype(jnp.float32)
            cnt = jax.lax.dot_general(
                jnp.ones((1, V), jnp.float32),
                eq,
                (((1,), (0,)), ((), ())),
                preferred_element_type=jnp.float32,
            )
            key = (
                cnt.reshape(SUB, LANE) * Y
                + sig_ref[0, i].astype(jnp.float32)
                - (ncon_ref[g * BB + i] * Y).astype(jnp.float32)
            )
            hist = jnp.stack(
                [jnp.sum((key == c).astype(jnp.float32)) for c in range(Y)]
            ).reshape(1, Y)
            z = jnp.maximum(jnp.sum(hist), 1.0)
            out_ref[0, i] = jnp.log(jnp.maximum(hist / z, 1e-9))

    out = pl.pallas_call(
        body,
        grid_spec=grid_spec,
        out_shape=jax.ShapeDtypeStruct((B // BB, BB, 1, Y), jnp.float32),
        compiler_params=pltpu.CompilerParams(
            dimension_semantics=("arbitrary",)
        ),
    )(ncon, t4, req3, s3)
    return out.reshape(B, Y)


# speedup vs baseline: 1.4132x; 1.0741x over previous
import jax
import jax.numpy as jnp
from jax.experimental import pallas as pl
from jax.experimental.pallas import tpu as pltpu

Y = 64
SUB, LANE = 256, 128
BB = 4


def kernel(tables, sigma, base_obs, actions, responses, t):
    B, V, N = tables.shape
    T = actions.shape[1]
    assert N == SUB * LANE and B % BB == 0

    actions = actions.astype(jnp.int32)
    responses = responses.astype(jnp.int32)
    base_obs = base_obs.astype(jnp.int32)

    active = jnp.arange(T, dtype=jnp.int32)[None, :] < t
    use_real = active & (actions != V)
    a_c = jnp.clip(actions, 0, V - 1)
    rows = jnp.concatenate(
        [jnp.zeros((B, 1), jnp.int32), jnp.where(use_real, a_c, 0)], axis=1
    )
    vals = jnp.concatenate(
        [base_obs[:, None], jnp.where(use_real, responses, base_obs[:, None])],
        axis=1,
    )

    BIG = jnp.int32(1 << 20)
    hit = rows[:, None, :] == jnp.arange(V, dtype=jnp.int32)[None, :, None]
    vmin = jnp.min(jnp.where(hit, vals[:, None, :], BIG), axis=2)
    vmax = jnp.max(jnp.where(hit, vals[:, None, :], -BIG), axis=2)
    con = jnp.any(hit, axis=2)
    req = jnp.where(con, vmin, -1).astype(jnp.int32)
    feasible = jnp.all(~con | (vmin == vmax), axis=1)
    ncon = jnp.where(
        feasible, jnp.sum(con.astype(jnp.int32), axis=1), V + 1
    ).astype(jnp.int32)

    s3 = sigma.reshape(B // BB, BB, SUB, LANE)
    req3 = req.reshape(B // BB, BB, V)[:, :, :, None]
    t4 = tables.reshape(B // BB, BB, V, N)

    grid_spec = pltpu.PrefetchScalarGridSpec(
        num_scalar_prefetch=1,
        grid=(B // BB,),
        in_specs=(
            pl.BlockSpec((1, BB, V, N), lambda g, s: (g, 0, 0, 0)),
            pl.BlockSpec((1, BB, V, 1), lambda g, s: (g, 0, 0, 0)),
            pl.BlockSpec((1, BB, SUB, LANE), lambda g, s: (g, 0, 0, 0)),
        ),
        out_specs=pl.BlockSpec((1, BB, 1, Y), lambda g, s: (g, 0, 0, 0)),
    )

    def body(ncon_ref, tab_ref, req_ref, sig_ref, out_ref):
        g = pl.program_id(0)
        for i in range(BB):
            tab = tab_ref[0, i]
            req = req_ref[0, i]
            eq = (tab == req).astype(jnp.float32)
            cnt = jax.lax.dot_general(
                jnp.ones((1, V), jnp.float32),
                eq,
                (((1,), (0,)), ((), ())),
                preferred_element_type=jnp.float32,
            )
            key = (
                cnt.reshape(SUB, LANE) * Y
                + sig_ref[0, i].astype(jnp.float32)
                - (ncon_ref[g * BB + i] * Y).astype(jnp.float32)
            )
            hist = jnp.stack(
                [jnp.sum((key == c).astype(jnp.float32)) for c in range(Y)]
            ).reshape(1, Y)
            z = jnp.maximum(jnp.sum(hist), 1.0)
            out_ref[0, i] = jnp.log(jnp.maximum(hist / z, 1e-9))

    out = pl.pallas_call(
        body,
        grid_spec=grid_spec,
        out_shape=jax.ShapeDtypeStruct((B // BB, BB, 1, Y), jnp.float32),
        compiler_params=pltpu.CompilerParams(
            dimension_semantics=("arbitrary",)
        ),
    )(ncon, t4, req3, s3)
    return out.reshape(B, Y)
